# baseline (speedup 1.0000x reference)
import jax
import jax.numpy as jnp
from jax import lax
from jax.experimental import pallas as pl
from jax.experimental.pallas import tpu as pltpu

PRE_NMS_TOP_N = 5000
POST_NMS_TOP_N = 2000
NMS_THRESH = 0.7
NMS_AUG_THICKNESS = 0.2
MIN_SIZE = 0.01

_B = 256
_NP = 5120
_NB = _NP // _B


def _nms_block_kernel(a_ref, r_ref, s_ref, prop_ref, masked_ref,
                      der_ref, supp_ref, cmp_ref):
    b = pl.program_id(0)
    boff = b * _B

    @pl.when(b == 0)
    def _init():
        xa = a_ref[0:1, :]
        ya = a_ref[1:2, :]
        za = a_ref[2:3, :]
        wa = a_ref[3:4, :]
        la = a_ref[4:5, :]
        ha = a_ref[5:6, :]
        ra = a_ref[6:7, :]
        tx = r_ref[0:1, :]
        ty = r_ref[1:2, :]
        tz = r_ref[2:3, :]
        tw = r_ref[3:4, :]
        tl = r_ref[4:5, :]
        th = r_ref[5:6, :]
        tr = r_ref[6:7, :]
        diag = jnp.sqrt(wa * wa + la * la)
        x = tx * diag + xa
        y = ty * diag + ya
        z = tz * ha + za
        w = jnp.exp(tw) * wa
        l = jnp.exp(tl) * la
        h = jnp.exp(th) * ha
        r = tr + ra
        prop_ref[0:1, :] = x
        prop_ref[1:2, :] = y
        prop_ref[2:3, :] = z
        prop_ref[3:4, :] = w
        prop_ref[4:5, :] = l
        prop_ref[5:6, :] = h
        prop_ref[6:7, :] = r
        prop_ref[7:8, :] = jnp.zeros_like(x)
        h_eff = jnp.maximum(h, NMS_AUG_THICKNESS)
        der_ref[0:1, :] = x - w * 0.5
        der_ref[1:2, :] = x + w * 0.5
        der_ref[2:3, :] = y - l * 0.5
        der_ref[3:4, :] = y + l * 0.5
        der_ref[4:5, :] = z
        der_ref[5:6, :] = z + h_eff
        der_ref[6:7, :] = w * l * h_eff
        der_ref[7:8, :] = jnp.zeros_like(x)
        valid = ((w >= MIN_SIZE) & (l >= MIN_SIZE) & (h >= MIN_SIZE))
        supp_ref[...] = jnp.where(valid, 0.0, 1.0)

    rows_blk = der_ref[:, pl.ds(boff, _B)]
    ri = lax.broadcasted_iota(jnp.int32, (_B, _B), 0)
    ci = lax.broadcasted_iota(jnp.int32, (_B, _B), 1)
    ident = (ri == ci).astype(jnp.float32)
    cols_blk = lax.dot_general(ident, rows_blk,
                               (((1,), (1,)), ((), ())),
                               preferred_element_type=jnp.float32)

    xlo_c = cols_blk[:, 0:1]
    xhi_c = cols_blk[:, 1:2]
    ylo_c = cols_blk[:, 2:3]
    yhi_c = cols_blk[:, 3:4]
    zlo_c = cols_blk[:, 4:5]
    zhi_c = cols_blk[:, 5:6]
    vol_c = cols_blk[:, 6:7]

    xlo_r = der_ref[0:1, :]
    xhi_r = der_ref[1:2, :]
    ylo_r = der_ref[2:3, :]
    yhi_r = der_ref[3:4, :]
    zlo_r = der_ref[4:5, :]
    zhi_r = der_ref[5:6, :]
    vol_r = der_ref[6:7, :]

    ox = jnp.maximum(0.0, jnp.minimum(xhi_c, xhi_r) - jnp.maximum(xlo_c, xlo_r))
    oy = jnp.maximum(0.0, jnp.minimum(yhi_c, yhi_r) - jnp.maximum(ylo_c, ylo_r))
    oz = jnp.maximum(0.0, jnp.minimum(zhi_c, zhi_r) - jnp.maximum(zlo_c, zlo_r))
    inter = ox * oy * oz
    union = vol_c + vol_r - inter
    cmp = (inter > NMS_THRESH * (union + 1e-6)).astype(jnp.float32)
    cmp_ref[...] = cmp

    suppb0 = supp_ref[0:1, pl.ds(boff, _B)]
    keepb0 = jnp.zeros_like(suppb0)
    iota_l = lax.broadcasted_iota(jnp.int32, (1, _B), 1)

    def body(i, carry):
        suppb, keepb = carry
        sel = (iota_l == i).astype(jnp.float32)
        ok = 1.0 - jnp.max(sel * suppb)
        keepb = keepb + sel * ok
        rowi = cmp_ref[pl.ds(i, 1), pl.ds(boff, _B)]
        suppb = jnp.maximum(suppb, ok * rowi)
        return suppb, keepb

    _, keep_f = lax.fori_loop(0, _B, body, (suppb0, keepb0))

    contrib = jnp.dot(keep_f, cmp, preferred_element_type=jnp.float32)
    supp_ref[...] = jnp.maximum(supp_ref[...],
                                jnp.where(contrib > 0.5, 1.0, 0.0))

    masked_ref[0:1, pl.ds(boff, _B)] = jnp.where(
        keep_f > 0.5, s_ref[0:1, pl.ds(boff, _B)], -1.0)


def _run_nms(a_t, r_t, s_p, interpret=False):
    return pl.pallas_call(
        _nms_block_kernel,
        grid=(_NB,),
        in_specs=[
            pl.BlockSpec((8, _NP), lambda b: (0, 0)),
            pl.BlockSpec((8, _NP), lambda b: (0, 0)),
            pl.BlockSpec((1, _NP), lambda b: (0, 0)),
        ],
        out_specs=[
            pl.BlockSpec((8, _NP), lambda b: (0, 0)),
            pl.BlockSpec((1, _NP), lambda b: (0, 0)),
        ],
        out_shape=[
            jax.ShapeDtypeStruct((8, _NP), jnp.float32),
            jax.ShapeDtypeStruct((1, _NP), jnp.float32),
        ],
        scratch_shapes=[
            pltpu.VMEM((8, _NP), jnp.float32),
            pltpu.VMEM((1, _NP), jnp.float32),
            pltpu.VMEM((_B, _NP), jnp.float32),
        ],
        interpret=interpret,
    )(a_t, r_t, s_p)


def kernel(anchors, objectness, box_regression):
    scores = jax.nn.sigmoid(objectness)
    top_scores, top_idx = lax.top_k(scores, PRE_NMS_TOP_N)
    a = anchors[top_idx]
    r = box_regression[top_idx]

    a_t = jnp.zeros((8, _NP), jnp.float32).at[:7, :PRE_NMS_TOP_N].set(a.T)
    r_t = jnp.zeros((8, _NP), jnp.float32).at[:7, :PRE_NMS_TOP_N].set(r.T)
    s_p = jnp.full((1, _NP), -1.0, jnp.float32).at[0, :PRE_NMS_TOP_N].set(top_scores)

    prop_t, masked = _run_nms(a_t, r_t, s_p)

    masked_scores = masked[0, :PRE_NMS_TOP_N]
    keep = masked_scores > -0.5
    kc = jnp.cumsum(keep.astype(jnp.int32))
    nk = kc[-1]
    ar = jnp.arange(PRE_NMS_TOP_N, dtype=jnp.int32)
    rank = jnp.where(keep, kc - 1, nk + ar - kc)
    post_idx = jnp.zeros((PRE_NMS_TOP_N,), jnp.int32).at[rank].set(ar)[
        :POST_NMS_TOP_N]
    post_scores = masked_scores[post_idx]
    proposals = prop_t[:7, :PRE_NMS_TOP_N].T
    post_boxes = proposals[post_idx]
    return jnp.concatenate([post_boxes, post_scores[:, None]], axis=1)


# speedup vs baseline: 1.0201x; 1.0201x over previous
import jax
import jax.numpy as jnp
from jax import lax
from jax.experimental import pallas as pl
from jax.experimental.pallas import tpu as pltpu

PRE_NMS_TOP_N = 5000
POST_NMS_TOP_N = 2000
NMS_THRESH = 0.7
NMS_AUG_THICKNESS = 0.2
MIN_SIZE = 0.01

_B = 256
_NP = 5120
_NB = _NP // _B


def _nms_block_kernel(a_ref, r_ref, s_ref, prop_ref, masked_ref,
                      der_ref, supp_ref, cmp_ref):
    b = pl.program_id(0)
    boff = b * _B

    @pl.when(b == 0)
    def _init():
        xa = a_ref[0:1, :]
        ya = a_ref[1:2, :]
        za = a_ref[2:3, :]
        wa = a_ref[3:4, :]
        la = a_ref[4:5, :]
        ha = a_ref[5:6, :]
        ra = a_ref[6:7, :]
        tx = r_ref[0:1, :]
        ty = r_ref[1:2, :]
        tz = r_ref[2:3, :]
        tw = r_ref[3:4, :]
        tl = r_ref[4:5, :]
        th = r_ref[5:6, :]
        tr = r_ref[6:7, :]
        diag = jnp.sqrt(wa * wa + la * la)
        x = tx * diag + xa
        y = ty * diag + ya
        z = tz * ha + za
        w = jnp.exp(tw) * wa
        l = jnp.exp(tl) * la
        h = jnp.exp(th) * ha
        r = tr + ra
        prop_ref[0:1, :] = x
        prop_ref[1:2, :] = y
        prop_ref[2:3, :] = z
        prop_ref[3:4, :] = w
        prop_ref[4:5, :] = l
        prop_ref[5:6, :] = h
        prop_ref[6:7, :] = r
        prop_ref[7:8, :] = jnp.zeros_like(x)
        h_eff = jnp.maximum(h, NMS_AUG_THICKNESS)
        der_ref[0:1, :] = x - w * 0.5
        der_ref[1:2, :] = x + w * 0.5
        der_ref[2:3, :] = y - l * 0.5
        der_ref[3:4, :] = y + l * 0.5
        der_ref[4:5, :] = z
        der_ref[5:6, :] = z + h_eff
        der_ref[6:7, :] = w * l * h_eff
        der_ref[7:8, :] = jnp.zeros_like(x)
        valid = ((w >= MIN_SIZE) & (l >= MIN_SIZE) & (h >= MIN_SIZE))
        supp_ref[...] = jnp.where(valid, 0.0, 1.0)

    rows_blk = der_ref[:, pl.ds(boff, _B)]
    ri = lax.broadcasted_iota(jnp.int32, (_B, _B), 0)
    ci = lax.broadcasted_iota(jnp.int32, (_B, _B), 1)
    ident = (ri == ci).astype(jnp.float32)
    cols_blk = lax.dot_general(ident, rows_blk,
                               (((1,), (1,)), ((), ())),
                               preferred_element_type=jnp.float32)

    xlo_c = cols_blk[:, 0:1]
    xhi_c = cols_blk[:, 1:2]
    ylo_c = cols_blk[:, 2:3]
    yhi_c = cols_blk[:, 3:4]
    zlo_c = cols_blk[:, 4:5]
    zhi_c = cols_blk[:, 5:6]
    vol_c = cols_blk[:, 6:7]

    xlo_r = der_ref[0:1, :]
    xhi_r = der_ref[1:2, :]
    ylo_r = der_ref[2:3, :]
    yhi_r = der_ref[3:4, :]
    zlo_r = der_ref[4:5, :]
    zhi_r = der_ref[5:6, :]
    vol_r = der_ref[6:7, :]

    ox = jnp.maximum(0.0, jnp.minimum(xhi_c, xhi_r) - jnp.maximum(xlo_c, xlo_r))
    oy = jnp.maximum(0.0, jnp.minimum(yhi_c, yhi_r) - jnp.maximum(ylo_c, ylo_r))
    oz = jnp.maximum(0.0, jnp.minimum(zhi_c, zhi_r) - jnp.maximum(zlo_c, zlo_r))
    inter = ox * oy * oz
    union = vol_c + vol_r - inter
    cmp = (inter > NMS_THRESH * (union + 1e-6)).astype(jnp.float32)
    cmp_ref[...] = cmp

    suppb0 = supp_ref[0:1, pl.ds(boff, _B)]
    keepb0 = jnp.zeros_like(suppb0)
    iota_l = lax.broadcasted_iota(jnp.int32, (1, _B), 1)

    def chunk_body(k, carry):
        suppb, keepb = carry
        chunk = cmp_ref[pl.ds(8 * k, 8), pl.ds(boff, _B)]
        for j in range(8):
            i = 8 * k + j
            sel = (iota_l == i).astype(jnp.float32)
            ok = 1.0 - jnp.max(sel * suppb)
            keepb = keepb + sel * ok
            suppb = jnp.maximum(suppb, ok * chunk[j:j + 1, :])
        return suppb, keepb

    _, keep_f = lax.fori_loop(0, _B // 8, chunk_body, (suppb0, keepb0))

    contrib = jnp.dot(keep_f, cmp, preferred_element_type=jnp.float32)
    supp_ref[...] = jnp.maximum(supp_ref[...],
                                jnp.where(contrib > 0.5, 1.0, 0.0))

    masked_ref[0:1, pl.ds(boff, _B)] = jnp.where(
        keep_f > 0.5, s_ref[0:1, pl.ds(boff, _B)], -1.0)


def _run_nms(a_t, r_t, s_p, interpret=False):
    return pl.pallas_call(
        _nms_block_kernel,
        grid=(_NB,),
        in_specs=[
            pl.BlockSpec((8, _NP), lambda b: (0, 0)),
            pl.BlockSpec((8, _NP), lambda b: (0, 0)),
            pl.BlockSpec((1, _NP), lambda b: (0, 0)),
        ],
        out_specs=[
            pl.BlockSpec((8, _NP), lambda b: (0, 0)),
            pl.BlockSpec((1, _NP), lambda b: (0, 0)),
        ],
        out_shape=[
            jax.ShapeDtypeStruct((8, _NP), jnp.float32),
            jax.ShapeDtypeStruct((1, _NP), jnp.float32),
        ],
        scratch_shapes=[
            pltpu.VMEM((8, _NP), jnp.float32),
            pltpu.VMEM((1, _NP), jnp.float32),
            pltpu.VMEM((_B, _NP), jnp.float32),
        ],
        interpret=interpret,
    )(a_t, r_t, s_p)


def kernel(anchors, objectness, box_regression):
    scores = jax.nn.sigmoid(objectness)
    top_scores, top_idx = lax.top_k(scores, PRE_NMS_TOP_N)
    a = anchors[top_idx]
    r = box_regression[top_idx]

    a_t = jnp.zeros((8, _NP), jnp.float32).at[:7, :PRE_NMS_TOP_N].set(a.T)
    r_t = jnp.zeros((8, _NP), jnp.float32).at[:7, :PRE_NMS_TOP_N].set(r.T)
    s_p = jnp.full((1, _NP), -1.0, jnp.float32).at[0, :PRE_NMS_TOP_N].set(top_scores)

    prop_t, masked = _run_nms(a_t, r_t, s_p)

    masked_scores = masked[0, :PRE_NMS_TOP_N]
    post_scores, post_idx = lax.top_k(masked_scores, POST_NMS_TOP_N)
    proposals = prop_t[:7, :PRE_NMS_TOP_N].T
    post_boxes = proposals[post_idx]
    return jnp.concatenate([post_boxes, post_scores[:, None]], axis=1)


# speedup vs baseline: 1.3599x; 1.3331x over previous
import jax
import jax.numpy as jnp
from jax import lax
from jax.experimental import pallas as pl
from jax.experimental.pallas import tpu as pltpu

PRE_NMS_TOP_N = 5000
POST_NMS_TOP_N = 2000
NMS_THRESH = 0.7
NMS_AUG_THICKNESS = 0.2
MIN_SIZE = 0.01

_B = 256
_NP = 5120
_NB = _NP // _B


def _nms_block_kernel(a_ref, r_ref, s_ref, prop_ref, masked_ref,
                      der_ref, supp_ref, cmp_ref):
    b = pl.program_id(0)
    boff = b * _B

    @pl.when(b == 0)
    def _init():
        xa = a_ref[0:1, :]
        ya = a_ref[1:2, :]
        za = a_ref[2:3, :]
        wa = a_ref[3:4, :]
        la = a_ref[4:5, :]
        ha = a_ref[5:6, :]
        ra = a_ref[6:7, :]
        tx = r_ref[0:1, :]
        ty = r_ref[1:2, :]
        tz = r_ref[2:3, :]
        tw = r_ref[3:4, :]
        tl = r_ref[4:5, :]
        th = r_ref[5:6, :]
        tr = r_ref[6:7, :]
        diag = jnp.sqrt(wa * wa + la * la)
        x = tx * diag + xa
        y = ty * diag + ya
        z = tz * ha + za
        w = jnp.exp(tw) * wa
        l = jnp.exp(tl) * la
        h = jnp.exp(th) * ha
        r = tr + ra
        prop_ref[0:1, :] = x
        prop_ref[1:2, :] = y
        prop_ref[2:3, :] = z
        prop_ref[3:4, :] = w
        prop_ref[4:5, :] = l
        prop_ref[5:6, :] = h
        prop_ref[6:7, :] = r
        prop_ref[7:8, :] = jnp.zeros_like(x)
        h_eff = jnp.maximum(h, NMS_AUG_THICKNESS)
        der_ref[0:1, :] = x - w * 0.5
        der_ref[1:2, :] = x + w * 0.5
        der_ref[2:3, :] = y - l * 0.5
        der_ref[3:4, :] = y + l * 0.5
        der_ref[4:5, :] = z
        der_ref[5:6, :] = z + h_eff
        der_ref[6:7, :] = w * l * h_eff
        der_ref[7:8, :] = jnp.zeros_like(x)
        valid = ((w >= MIN_SIZE) & (l >= MIN_SIZE) & (h >= MIN_SIZE))
        supp_ref[...] = jnp.where(valid, 0.0, 1.0)

    rows_blk = der_ref[:, pl.ds(boff, _B)]
    ri = lax.broadcasted_iota(jnp.int32, (_B, _B), 0)
    ci = lax.broadcasted_iota(jnp.int32, (_B, _B), 1)
    ident = (ri == ci).astype(jnp.float32)
    cols_blk = lax.dot_general(ident, rows_blk,
                               (((1,), (1,)), ((), ())),
                               preferred_element_type=jnp.float32)

    xlo_c = cols_blk[:, 0:1]
    xhi_c = cols_blk[:, 1:2]
    ylo_c = cols_blk[:, 2:3]
    yhi_c = cols_blk[:, 3:4]
    zlo_c = cols_blk[:, 4:5]
    zhi_c = cols_blk[:, 5:6]
    vol_c = cols_blk[:, 6:7]

    xlo_r = der_ref[0:1, :]
    xhi_r = der_ref[1:2, :]
    ylo_r = der_ref[2:3, :]
    yhi_r = der_ref[3:4, :]
    zlo_r = der_ref[4:5, :]
    zhi_r = der_ref[5:6, :]
    vol_r = der_ref[6:7, :]

    ox = jnp.maximum(0.0, jnp.minimum(xhi_c, xhi_r) - jnp.maximum(xlo_c, xlo_r))
    oy = jnp.maximum(0.0, jnp.minimum(yhi_c, yhi_r) - jnp.maximum(ylo_c, ylo_r))
    oz = jnp.maximum(0.0, jnp.minimum(zhi_c, zhi_r) - jnp.maximum(zlo_c, zlo_r))
    inter = ox * oy * oz
    union = vol_c + vol_r - inter
    cmp = (inter > NMS_THRESH * (union + 1e-6)).astype(jnp.float32)
    cmp_ref[...] = cmp

    suppb0 = supp_ref[0:1, pl.ds(boff, _B)]
    keepb0 = jnp.zeros_like(suppb0)
    iota_l = lax.broadcasted_iota(jnp.int32, (1, _B), 1)

    def chunk_body(k, carry):
        suppb, keepb = carry
        chunk = cmp_ref[pl.ds(8 * k, 8), pl.ds(boff, _B)]
        for j in range(8):
            i = 8 * k + j
            sel = (iota_l == i).astype(jnp.float32)
            ok = 1.0 - jnp.max(sel * suppb, axis=1, keepdims=True)
            keepb = keepb + sel * ok
            suppb = jnp.maximum(suppb, ok * chunk[j:j + 1, :])
        return suppb, keepb

    _, keep_f = lax.fori_loop(0, _B // 8, chunk_body, (suppb0, keepb0))

    contrib = jnp.dot(keep_f, cmp, preferred_element_type=jnp.float32)
    supp_ref[...] = jnp.maximum(supp_ref[...],
                                jnp.where(contrib > 0.5, 1.0, 0.0))

    masked_ref[0:1, pl.ds(boff, _B)] = jnp.where(
        keep_f > 0.5, s_ref[0:1, pl.ds(boff, _B)], -1.0)


def _run_nms(a_t, r_t, s_p, interpret=False):
    return pl.pallas_call(
        _nms_block_kernel,
        grid=(_NB,),
        in_specs=[
            pl.BlockSpec((8, _NP), lambda b: (0, 0)),
            pl.BlockSpec((8, _NP), lambda b: (0, 0)),
            pl.BlockSpec((1, _NP), lambda b: (0, 0)),
        ],
        out_specs=[
            pl.BlockSpec((8, _NP), lambda b: (0, 0)),
            pl.BlockSpec((1, _NP), lambda b: (0, 0)),
        ],
        out_shape=[
            jax.ShapeDtypeStruct((8, _NP), jnp.float32),
            jax.ShapeDtypeStruct((1, _NP), jnp.float32),
        ],
        scratch_shapes=[
            pltpu.VMEM((8, _NP), jnp.float32),
            pltpu.VMEM((1, _NP), jnp.float32),
            pltpu.VMEM((_B, _NP), jnp.float32),
        ],
        interpret=interpret,
    )(a_t, r_t, s_p)


def kernel(anchors, objectness, box_regression):
    scores = jax.nn.sigmoid(objectness)
    top_scores, top_idx = lax.top_k(scores, PRE_NMS_TOP_N)
    a = anchors[top_idx]
    r = box_regression[top_idx]

    a_t = jnp.zeros((8, _NP), jnp.float32).at[:7, :PRE_NMS_TOP_N].set(a.T)
    r_t = jnp.zeros((8, _NP), jnp.float32).at[:7, :PRE_NMS_TOP_N].set(r.T)
    s_p = jnp.full((1, _NP), -1.0, jnp.float32).at[0, :PRE_NMS_TOP_N].set(top_scores)

    prop_t, masked = _run_nms(a_t, r_t, s_p)

    masked_scores = masked[0, :PRE_NMS_TOP_N]
    post_scores, post_idx = lax.top_k(masked_scores, POST_NMS_TOP_N)
    proposals = prop_t[:7, :PRE_NMS_TOP_N].T
    post_boxes = proposals[post_idx]
    return jnp.concatenate([post_boxes, post_scores[:, None]], axis=1)
